# baseline (speedup 1.0000x reference)
import functools

import jax
import jax.numpy as jnp
import numpy as np
from jax import lax
from jax.experimental import pallas as pl
from jax.experimental.pallas import tpu as pltpu
from jax.experimental.pallas import tpu_sc as plsc

_NUM_FIELDS = 26
_FIELD_SIZE = 100000
_BATCH = 4096
_DIM = 64
_ROWS = _NUM_FIELDS * _FIELD_SIZE

_NC = 2
_NS = 16
_NW = _NC * _NS

_DPW = _DIM // _NW
_SUB = 128


@functools.partial(
    pl.kernel,
    mesh=plsc.VectorSubcoreMesh(core_axis_name="c", subcore_axis_name="s"),
    out_type=jax.ShapeDtypeStruct((_NUM_FIELDS, _DIM, _BATCH), jnp.float32),
    scratch_types=[
        pltpu.VMEM((_BATCH,), jnp.int32),
        pltpu.VMEM((2, _BATCH), jnp.float32),
        pltpu.SemaphoreType.DMA,
        pltpu.SemaphoreType.DMA,
    ],
    compiler_params=pltpu.CompilerParams(use_tc_tiling_on_sc=False),
)
def _sc_gather(g_hbm, tt1_hbm, out_hbm, g_v, plane_v, sem_g, sem_w):
    wid = lax.axis_index("s") * _NC + lax.axis_index("c")

    def step(i, b):
        d = wid * _DPW + i // _NUM_FIELDS
        f = lax.rem(i, _NUM_FIELDS)
        pltpu.sync_copy(g_hbm.at[d, f], g_v)

        def fire(k, carry):
            pltpu.async_copy(
                tt1_hbm.at[g_v.at[pl.ds(k * _SUB, _SUB)]],
                plane_v.at[b, pl.ds(k * _SUB, _SUB)],
                sem_g,
            )
            return carry

        lax.fori_loop(0, _BATCH // _SUB, fire, 0)
        pltpu.make_async_copy(
            out_hbm.at[f, d], plane_v.at[b], sem_g
        ).wait()
        pltpu.async_copy(plane_v.at[b], out_hbm.at[f, d], sem_w)

    def body(g2, carry):
        for b in range(2):
            i = g2 * 2 + b
            step(i, b)

            @pl.when(i >= 1)
            def _():
                pltpu.make_async_copy(
                    plane_v.at[1 - b], out_hbm.at[0, 0], sem_w
                ).wait()

        return carry

    lax.fori_loop(0, _DPW * _NUM_FIELDS // 2, body, 0)
    pltpu.make_async_copy(plane_v.at[1], out_hbm.at[0, 0], sem_w).wait()


def kernel(x, table):
    offs = jnp.asarray(np.arange(_NUM_FIELDS) * _FIELD_SIZE, dtype=jnp.int32)
    idx_t = (x + offs[None, :]).T
    g = (jnp.arange(_DIM, dtype=jnp.int32) * _ROWS)[:, None, None] + idx_t
    tt1 = table.T.reshape(-1)
    out = _sc_gather(g, tt1)
    return jnp.transpose(out, (2, 0, 1))


# speedup vs baseline: 13.4007x; 13.4007x over previous
import functools

import jax
import jax.numpy as jnp
import numpy as np
from jax import lax
from jax.experimental import pallas as pl
from jax.experimental.pallas import tpu as pltpu
from jax.experimental.pallas import tpu_sc as plsc

_NUM_FIELDS = 26
_FIELD_SIZE = 100000
_BATCH = 4096
_DIM = 64

_NC = 2
_NS = 16
_NW = _NC * _NS

_N = _BATCH * _NUM_FIELDS
_PER_W = _N // _NW
_CHUNK = 128
_NCH = _PER_W // _CHUNK


@functools.partial(
    pl.kernel,
    mesh=plsc.VectorSubcoreMesh(core_axis_name="c", subcore_axis_name="s"),
    out_type=jax.ShapeDtypeStruct((_N // 2, 2 * _DIM), jnp.float32),
    scratch_types=[
        pltpu.VMEM((_PER_W,), jnp.int32),
        pltpu.VMEM((2, _CHUNK // 2, 2 * _DIM), jnp.float32),
        pltpu.SemaphoreType.DMA,
        pltpu.SemaphoreType.DMA,
    ],
    compiler_params=pltpu.CompilerParams(use_tc_tiling_on_sc=True),
)
def _sc_gather(idx_hbm, table_hbm, out_hbm, idx_v, rows_v, sem_g, sem_w):
    wid = lax.axis_index("s") * _NC + lax.axis_index("c")
    base2 = wid * (_PER_W // 2)
    pltpu.sync_copy(idx_hbm.at[wid], idx_v)

    def fire_chunk(c, b):
        def block(jb, carry):
            rv = idx_v[pl.ds(c * _CHUNK + jb * 16, 16)]
            for rr in range(16):
                j = jb * 16 + rr
                pltpu.async_copy(
                    table_hbm.at[rv[rr]],
                    rows_v.at[b, j // 2, pl.ds((rr % 2) * _DIM, _DIM)],
                    sem_g,
                )
            return carry

        lax.fori_loop(0, _CHUNK // 16, block, 0)

    def drain_chunk(b):
        pltpu.make_async_copy(
            out_hbm.at[pl.ds(base2, _CHUNK // 2)], rows_v.at[b], sem_g
        ).wait()

    def wout_start(c, b):
        pltpu.async_copy(
            rows_v.at[b],
            out_hbm.at[pl.ds(base2 + c * (_CHUNK // 2), _CHUNK // 2)],
            sem_w,
        )

    def wout_wait(b):
        pltpu.make_async_copy(
            rows_v.at[b], out_hbm.at[pl.ds(base2, _CHUNK // 2)], sem_w
        ).wait()

    fire_chunk(0, 0)

    def body(g, carry):
        for b in range(2):
            c = g * 2 + b
            drain_chunk(b)
            wout_start(c, b)

            @pl.when(c >= 1)
            def _():
                wout_wait(1 - b)

            @pl.when(c < _NCH - 1)
            def _():
                fire_chunk(c + 1, 1 - b)

        return carry

    lax.fori_loop(0, _NCH // 2, body, 0)
    wout_wait((_NCH - 1) % 2)


def kernel(x, table):
    offs = jnp.asarray(np.arange(_NUM_FIELDS) * _FIELD_SIZE, dtype=jnp.int32)
    idx = (x + offs[None, :]).reshape(_NW, _PER_W)
    out = _sc_gather(idx, table)
    return out.reshape(_BATCH, _NUM_FIELDS, _DIM)
